# Initial kernel scaffold; baseline (speedup 1.0000x reference)
import jax
import jax.numpy as jnp
from jax.experimental import pallas as pl


def kernel(x, W_lin, b_lin, emb, Wl1, bl1, Wr1, Wl2, bl2, Wr2, edge_index, node_ids, edge_label_index):
    raise NotImplementedError("write your pallas kernel here")


# speedup vs baseline: 2.6279x; 2.6279x over previous
import dataclasses
import functools

import jax
import jax.numpy as jnp
from jax import lax
from jax.experimental import pallas as pl
from jax.experimental.pallas import tpu as pltpu
from jax.experimental.pallas import tpu_sc as plsc

N = 10000
E = 320000
L = 100000
D = 128

NC = 2
NS = 16
NW = NC * NS

N2 = 10240
RPT = N2 // NS
EC = 80
EPW = E // NW
ECHUNKS = EPW // EC

CC = 128
CPW = 25
LP = NW * CPW * CC

_mesh = functools.partial(
    plsc.VectorSubcoreMesh, core_axis_name="c", subcore_axis_name="s")


def _sc_params():
  cp = pltpu.CompilerParams()
  if "needs_layout_passes" in pltpu.CompilerParams.__dataclass_fields__:
    cp = dataclasses.replace(cp, needs_layout_passes=False)
  return cp


def _sc_degree(dst):
  @functools.partial(
      pl.kernel,
      out_type=jax.ShapeDtypeStruct((NC, N2, 16), jnp.float32),
      mesh=_mesh(),
      compiler_params=_sc_params(),
      scratch_types=[
          pltpu.VMEM((EC,), jnp.int32),
          pltpu.VMEM((80, D), jnp.float32),
          pltpu.VMEM((80,), jnp.int32),
          pltpu.VMEM((RPT // D, D), jnp.float32),
          pltpu.VMEM((RPT, 16), jnp.float32),
          pltpu.VMEM_SHARED((80, D), jnp.float32),
      ],
  )
  def k(dst_hbm, out_hbm, didx, tab, rowids, degv, rowbuf, spacc):
    c = lax.axis_index("c")
    s = lax.axis_index("s")
    wid = c * NS + s
    z16 = jnp.zeros((16,), jnp.float32)

    @pl.loop(0, 80)
    def _(i):
      for q in range(D // 16):
        tab[i, pl.ds(q * 16, 16)] = z16

    @pl.loop(0, 80, step=16)
    def _(i):
      rowids[pl.ds(i, 16)] = lax.iota(jnp.int32, 16) + i

    @pl.loop(0, RPT // D)
    def _(i):
      for q in range(D // 16):
        degv[i, pl.ds(q * 16, 16)] = z16

    pltpu.sync_copy(degv, spacc.at[pl.ds(s * (RPT // D), RPT // D)])
    plsc.subcore_barrier()

    ones = jnp.ones((16,), jnp.float32)

    @pl.loop(0, ECHUNKS)
    def _(j):
      pltpu.sync_copy(dst_hbm.at[pl.ds(wid * EPW + j * EC, EC)], didx)
      for g in range(EC // 16):
        d = didx[pl.ds(g * 16, 16)]
        plsc.addupdate_scatter(
            tab, [lax.shift_right_logical(d, 7), d & 127], ones)

    pltpu.sync_copy(tab, spacc.at[rowids], add=True)
    plsc.subcore_barrier()

    pltpu.sync_copy(spacc.at[pl.ds(s * (RPT // D), RPT // D)], degv)

    @pl.loop(0, RPT)
    def _(i):
      iv = jnp.full((16,), i, jnp.int32)
      rowbuf[i, pl.ds(0, 16)] = plsc.load_gather(
          degv, [lax.shift_right_logical(iv, 7), iv & 127])

    pltpu.sync_copy(rowbuf, out_hbm.at[c].at[pl.ds(s * RPT, RPT)])

  return k(dst)


def _sc_aggregate(h, src, dst, zeros_nd):
  @functools.partial(
      pl.kernel,
      out_type=jax.ShapeDtypeStruct((NC, N2, D), jnp.float32),
      mesh=_mesh(),
      scratch_types=[
          pltpu.VMEM((EC,), jnp.int32),
          pltpu.VMEM((EC,), jnp.int32),
          pltpu.VMEM((EC, D), jnp.float32),
          pltpu.VMEM_SHARED((N2, D), jnp.float32),
      ],
  )
  def k(h_hbm, src_hbm, dst_hbm, z_hbm, out_hbm, sidx, didx, rows, acc):
    c = lax.axis_index("c")
    s = lax.axis_index("s")
    wid = c * NS + s
    pltpu.sync_copy(z_hbm.at[pl.ds(s * RPT, RPT)], acc.at[pl.ds(s * RPT, RPT)])
    plsc.subcore_barrier()

    @pl.loop(0, ECHUNKS)
    def _(j):
      base = wid * EPW + j * EC
      pltpu.sync_copy(src_hbm.at[pl.ds(base, EC)], sidx)
      pltpu.sync_copy(dst_hbm.at[pl.ds(base, EC)], didx)
      pltpu.sync_copy(h_hbm.at[sidx], rows)
      pltpu.sync_copy(rows, acc.at[didx], add=True)

    plsc.subcore_barrier()
    pltpu.sync_copy(acc.at[pl.ds(s * RPT, RPT)],
                    out_hbm.at[c].at[pl.ds(s * RPT, RPT)])

  return k(h, src, dst, zeros_nd)


def _sc_classify(h, si, di):
  @functools.partial(
      pl.kernel,
      out_type=jax.ShapeDtypeStruct((LP,), jnp.float32),
      mesh=_mesh(),
      compiler_params=_sc_params(),
      scratch_types=[
          pltpu.VMEM((CC,), jnp.int32),
          pltpu.VMEM((CC,), jnp.int32),
          pltpu.VMEM((CC, D), jnp.float32),
          pltpu.VMEM((CC, D), jnp.float32),
          pltpu.VMEM((CC,), jnp.float32),
      ],
  )
  def k(h_hbm, si_hbm, di_hbm, out_hbm, sidx, didx, arows, brows, ovec):
    c = lax.axis_index("c")
    s = lax.axis_index("s")
    wid = c * NS + s

    @pl.loop(0, CPW)
    def _(j):
      base = (wid * CPW + j) * CC
      pltpu.sync_copy(si_hbm.at[pl.ds(base, CC)], sidx)
      pltpu.sync_copy(di_hbm.at[pl.ds(base, CC)], didx)
      pltpu.sync_copy(h_hbm.at[sidx], arows)
      pltpu.sync_copy(h_hbm.at[didx], brows)

      @pl.loop(0, CC, step=16)
      def _(p):
        rids = lax.iota(jnp.int32, 16) + p
        acc = jnp.zeros((16,), jnp.float32)
        for q in range(D):
          cq = jnp.full((16,), q, jnp.int32)
          acc = acc + (plsc.load_gather(arows, [rids, cq]) *
                       plsc.load_gather(brows, [rids, cq]))
        ovec[pl.ds(p, 16)] = acc

      pltpu.sync_copy(ovec, out_hbm.at[pl.ds(base, CC)])

  return k(h, si, di)


def _tc_encode(x, w_t, b, emb):
  BM = 1000

  def body(x_ref, w_ref, b_ref, e_ref, o_ref):
    o_ref[...] = (
        jax.lax.dot(x_ref[...], w_ref[...],
                    precision=lax.Precision.HIGHEST,
                    preferred_element_type=jnp.float32)
        + b_ref[...] + e_ref[...])

  return pl.pallas_call(
      body,
      grid=(N // BM,),
      in_specs=[
          pl.BlockSpec((BM, D), lambda i: (i, 0)),
          pl.BlockSpec((D, D), lambda i: (0, 0)),
          pl.BlockSpec((1, D), lambda i: (0, 0)),
          pl.BlockSpec((BM, D), lambda i: (i, 0)),
      ],
      out_specs=pl.BlockSpec((BM, D), lambda i: (i, 0)),
      out_shape=jax.ShapeDtypeStruct((N, D), jnp.float32),
  )(x, w_t, b, emb)


def _tc_layer(p, degp, h_prev, wl_t, wr_t, bl, relu):
  BM = 1000

  def body(p_ref, d_ref, h_ref, wl_ref, wr_ref, b_ref, o_ref):
    agg = p_ref[0] + p_ref[1]
    deg = d_ref[0, :, 0:1] + d_ref[1, :, 0:1]
    mean = agg / jnp.maximum(deg, 1.0)
    out = (
        jax.lax.dot(mean, wl_ref[...], precision=lax.Precision.HIGHEST,
                    preferred_element_type=jnp.float32)
        + jax.lax.dot(h_ref[...], wr_ref[...],
                      precision=lax.Precision.HIGHEST,
                      preferred_element_type=jnp.float32)
        + b_ref[...])
    if relu:
      out = jnp.maximum(out, 0.0)
    o_ref[...] = out

  return pl.pallas_call(
      body,
      grid=(N // BM,),
      in_specs=[
          pl.BlockSpec((NC, BM, D), lambda i: (0, i, 0)),
          pl.BlockSpec((NC, BM, 16), lambda i: (0, i, 0)),
          pl.BlockSpec((BM, D), lambda i: (i, 0)),
          pl.BlockSpec((D, D), lambda i: (0, 0)),
          pl.BlockSpec((D, D), lambda i: (0, 0)),
          pl.BlockSpec((1, D), lambda i: (0, 0)),
      ],
      out_specs=pl.BlockSpec((BM, D), lambda i: (i, 0)),
      out_shape=jax.ShapeDtypeStruct((N, D), jnp.float32),
  )(p, degp, h_prev, wl_t, wr_t, bl)


def kernel(x, W_lin, b_lin, emb, Wl1, bl1, Wr1, Wl2, bl2, Wr2,
           edge_index, node_ids, edge_label_index):
  src = edge_index[0].astype(jnp.int32)
  dst = edge_index[1].astype(jnp.int32)
  eli = edge_label_index.astype(jnp.int32)
  si = jnp.pad(eli[0], (0, LP - L))
  di = jnp.pad(eli[1], (0, LP - L))

  zeros_nd = jnp.zeros((N2, D), jnp.float32)

  degp = _sc_degree(dst)
  h0 = _tc_encode(x, W_lin.T, b_lin.reshape(1, D), emb)

  p1 = _sc_aggregate(h0, src, dst, zeros_nd)
  h1 = _tc_layer(p1, degp, h0, Wl1.T, Wr1.T, bl1.reshape(1, D), relu=True)

  p2 = _sc_aggregate(h1, src, dst, zeros_nd)
  h2 = _tc_layer(p2, degp, h1, Wl2.T, Wr2.T, bl2.reshape(1, D), relu=False)

  pred = _sc_classify(h2, si, di)
  return pred[:L]


# speedup vs baseline: 4.7237x; 1.7975x over previous
import dataclasses
import functools

import jax
import jax.numpy as jnp
from jax import lax
from jax.experimental import pallas as pl
from jax.experimental.pallas import tpu as pltpu
from jax.experimental.pallas import tpu_sc as plsc

N = 10000
E = 320000
L = 100000
D = 128

NC = 2
NS = 16
NW = NC * NS

N2 = 10240
RPT = N2 // NS
EC = 80
EPW = E // NW
ECHUNKS = EPW // EC

CC = 128
CPW = 25
LP = NW * CPW * CC

_mesh = functools.partial(
    plsc.VectorSubcoreMesh, core_axis_name="c", subcore_axis_name="s")


def _sc_params():
  cp = pltpu.CompilerParams()
  if "needs_layout_passes" in pltpu.CompilerParams.__dataclass_fields__:
    cp = dataclasses.replace(cp, needs_layout_passes=False)
  return cp


def _sc_degree(dst):
  @functools.partial(
      pl.kernel,
      out_type=jax.ShapeDtypeStruct((NC, N2, 16), jnp.float32),
      mesh=_mesh(),
      compiler_params=_sc_params(),
      scratch_types=[
          pltpu.VMEM((EC,), jnp.int32),
          pltpu.VMEM((80, D), jnp.float32),
          pltpu.VMEM((80,), jnp.int32),
          pltpu.VMEM((RPT // D, D), jnp.float32),
          pltpu.VMEM((RPT, 16), jnp.float32),
          pltpu.VMEM_SHARED((80, D), jnp.float32),
      ],
  )
  def k(dst_hbm, out_hbm, didx, tab, rowids, degv, rowbuf, spacc):
    c = lax.axis_index("c")
    s = lax.axis_index("s")
    wid = c * NS + s
    z16 = jnp.zeros((16,), jnp.float32)

    @pl.loop(0, 80)
    def _(i):
      for q in range(D // 16):
        tab[i, pl.ds(q * 16, 16)] = z16

    @pl.loop(0, 80, step=16)
    def _(i):
      rowids[pl.ds(i, 16)] = lax.iota(jnp.int32, 16) + i

    @pl.loop(0, RPT // D)
    def _(i):
      for q in range(D // 16):
        degv[i, pl.ds(q * 16, 16)] = z16

    pltpu.sync_copy(degv, spacc.at[pl.ds(s * (RPT // D), RPT // D)])
    plsc.subcore_barrier()

    ones = jnp.ones((16,), jnp.float32)

    @pl.loop(0, ECHUNKS)
    def _(j):
      pltpu.sync_copy(dst_hbm.at[pl.ds(wid * EPW + j * EC, EC)], didx)
      for g in range(EC // 16):
        d = didx[pl.ds(g * 16, 16)]
        plsc.addupdate_scatter(
            tab, [lax.shift_right_logical(d, 7), d & 127], ones)

    pltpu.sync_copy(tab, spacc.at[rowids], add=True)
    plsc.subcore_barrier()

    pltpu.sync_copy(spacc.at[pl.ds(s * (RPT // D), RPT // D)], degv)

    @pl.loop(0, RPT)
    def _(i):
      iv = jnp.full((16,), i, jnp.int32)
      rowbuf[i, pl.ds(0, 16)] = plsc.load_gather(
          degv, [lax.shift_right_logical(iv, 7), iv & 127])

    pltpu.sync_copy(rowbuf, out_hbm.at[c].at[pl.ds(s * RPT, RPT)])

  return k(dst)


def _sc_aggregate(h, src, dst, zeros_nd):
  @functools.partial(
      pl.kernel,
      out_type=jax.ShapeDtypeStruct((NC, N2, D), jnp.float32),
      mesh=_mesh(),
      scratch_types=[
          pltpu.VMEM((EC,), jnp.int32),
          pltpu.VMEM((EC,), jnp.int32),
          pltpu.VMEM((EC, D), jnp.float32),
          pltpu.VMEM((EC,), jnp.int32),
          pltpu.VMEM((EC,), jnp.int32),
          pltpu.VMEM((EC, D), jnp.float32),
          pltpu.VMEM_SHARED((N2, D), jnp.float32),
          pltpu.SemaphoreType.DMA,
          pltpu.SemaphoreType.DMA,
      ],
  )
  def k(h_hbm, src_hbm, dst_hbm, z_hbm, out_hbm,
        sidxA, didxA, rowsA, sidxB, didxB, rowsB, acc, semA, semB):
    c = lax.axis_index("c")
    s = lax.axis_index("s")
    wid = c * NS + s
    pltpu.sync_copy(z_hbm.at[pl.ds(s * RPT, RPT)], acc.at[pl.ds(s * RPT, RPT)])
    plsc.subcore_barrier()

    def start(j, sidx, didx, rows, sem):
      base = wid * EPW + j * EC
      pltpu.sync_copy(src_hbm.at[pl.ds(base, EC)], sidx)
      pltpu.sync_copy(dst_hbm.at[pl.ds(base, EC)], didx)
      pltpu.async_copy(h_hbm.at[sidx], rows, sem)

    def finish(sidx, didx, rows, sem):
      pltpu.make_async_copy(h_hbm.at[sidx], rows, sem).wait()
      pltpu.sync_copy(rows, acc.at[didx], add=True)

    start(0, sidxA, didxA, rowsA, semA)

    @pl.loop(0, ECHUNKS - 1, step=2)
    def _(j):
      start(j + 1, sidxB, didxB, rowsB, semB)
      finish(sidxA, didxA, rowsA, semA)
      start(j + 2, sidxA, didxA, rowsA, semA)
      finish(sidxB, didxB, rowsB, semB)

    finish(sidxA, didxA, rowsA, semA)
    plsc.subcore_barrier()
    pltpu.sync_copy(acc.at[pl.ds(s * RPT, RPT)],
                    out_hbm.at[c].at[pl.ds(s * RPT, RPT)])

  return k(h, src, dst, zeros_nd)


def _sc_classify(h, si, di):
  @functools.partial(
      pl.kernel,
      out_type=jax.ShapeDtypeStruct((LP,), jnp.float32),
      mesh=_mesh(),
      compiler_params=_sc_params(),
      scratch_types=[
          pltpu.VMEM((CC,), jnp.int32),
          pltpu.VMEM((CC,), jnp.int32),
          pltpu.VMEM((CC, D), jnp.float32),
          pltpu.VMEM((CC, D), jnp.float32),
          pltpu.VMEM((CC,), jnp.int32),
          pltpu.VMEM((CC,), jnp.int32),
          pltpu.VMEM((CC, D), jnp.float32),
          pltpu.VMEM((CC, D), jnp.float32),
          pltpu.VMEM((CC,), jnp.float32),
          pltpu.VMEM((16 * 17,), jnp.float32),
          pltpu.SemaphoreType.DMA,
          pltpu.SemaphoreType.DMA,
      ],
  )
  def k(h_hbm, si_hbm, di_hbm, out_hbm,
        sidxA, didxA, arowsA, browsA,
        sidxB, didxB, arowsB, browsB,
        ovec, tile17, semA, semB):
    c = lax.axis_index("c")
    s = lax.axis_index("s")
    wid = c * NS + s
    lanes = lax.iota(jnp.int32, 16)

    def start(j, sidx, didx, arows, brows, sem):
      base = (wid * CPW + j) * CC
      pltpu.sync_copy(si_hbm.at[pl.ds(base, CC)], sidx)
      pltpu.sync_copy(di_hbm.at[pl.ds(base, CC)], didx)
      pltpu.async_copy(h_hbm.at[sidx], arows, sem)
      pltpu.async_copy(h_hbm.at[didx], brows, sem)

    def wait(sidx, didx, arows, brows, sem):
      pltpu.make_async_copy(h_hbm.at[sidx], arows, sem).wait()
      pltpu.make_async_copy(h_hbm.at[didx], brows, sem).wait()

    def compute(j, arows, brows):
      @pl.loop(0, CC, step=16)
      def _(p0):
        for p in range(16):
          pi = p0 + p
          acc = arows[pi, pl.ds(0, 16)] * brows[pi, pl.ds(0, 16)]
          for q in range(1, D // 16):
            acc += arows[pi, pl.ds(q * 16, 16)] * brows[pi, pl.ds(q * 16, 16)]
          plsc.store_scatter(tile17, [lanes + 17 * p], acc)
        sums = plsc.load_gather(tile17, [lanes * 17])
        for kcol in range(1, 16):
          sums += plsc.load_gather(tile17, [lanes * 17 + kcol])
        ovec[pl.ds(p0, 16)] = sums

      pltpu.sync_copy(ovec, out_hbm.at[pl.ds((wid * CPW + j) * CC, CC)])

    start(0, sidxA, didxA, arowsA, browsA, semA)

    @pl.loop(0, CPW - 1, step=2)
    def _(j):
      start(j + 1, sidxB, didxB, arowsB, browsB, semB)
      wait(sidxA, didxA, arowsA, browsA, semA)
      compute(j, arowsA, browsA)
      start(j + 2, sidxA, didxA, arowsA, browsA, semA)
      wait(sidxB, didxB, arowsB, browsB, semB)
      compute(j + 1, arowsB, browsB)

    wait(sidxA, didxA, arowsA, browsA, semA)
    compute(CPW - 1, arowsA, browsA)

  return k(h, si, di)


def _tc_encode(x, w_t, b, emb):
  BM = 1000

  def body(x_ref, w_ref, b_ref, e_ref, o_ref):
    o_ref[...] = (
        jax.lax.dot(x_ref[...], w_ref[...],
                    precision=lax.Precision.HIGHEST,
                    preferred_element_type=jnp.float32)
        + b_ref[...] + e_ref[...])

  return pl.pallas_call(
      body,
      grid=(N // BM,),
      in_specs=[
          pl.BlockSpec((BM, D), lambda i: (i, 0)),
          pl.BlockSpec((D, D), lambda i: (0, 0)),
          pl.BlockSpec((1, D), lambda i: (0, 0)),
          pl.BlockSpec((BM, D), lambda i: (i, 0)),
      ],
      out_specs=pl.BlockSpec((BM, D), lambda i: (i, 0)),
      out_shape=jax.ShapeDtypeStruct((N, D), jnp.float32),
  )(x, w_t, b, emb)


def _tc_layer(p, degp, h_prev, wl_t, wr_t, bl, relu):
  BM = 1000

  def body(p_ref, d_ref, h_ref, wl_ref, wr_ref, b_ref, o_ref):
    agg = p_ref[0] + p_ref[1]
    deg = d_ref[0, :, 0:1] + d_ref[1, :, 0:1]
    mean = agg / jnp.maximum(deg, 1.0)
    out = (
        jax.lax.dot(mean, wl_ref[...], precision=lax.Precision.HIGHEST,
                    preferred_element_type=jnp.float32)
        + jax.lax.dot(h_ref[...], wr_ref[...],
                      precision=lax.Precision.HIGHEST,
                      preferred_element_type=jnp.float32)
        + b_ref[...])
    if relu:
      out = jnp.maximum(out, 0.0)
    o_ref[...] = out

  return pl.pallas_call(
      body,
      grid=(N // BM,),
      in_specs=[
          pl.BlockSpec((NC, BM, D), lambda i: (0, i, 0)),
          pl.BlockSpec((NC, BM, 16), lambda i: (0, i, 0)),
          pl.BlockSpec((BM, D), lambda i: (i, 0)),
          pl.BlockSpec((D, D), lambda i: (0, 0)),
          pl.BlockSpec((D, D), lambda i: (0, 0)),
          pl.BlockSpec((1, D), lambda i: (0, 0)),
      ],
      out_specs=pl.BlockSpec((BM, D), lambda i: (i, 0)),
      out_shape=jax.ShapeDtypeStruct((N, D), jnp.float32),
  )(p, degp, h_prev, wl_t, wr_t, bl)


def kernel(x, W_lin, b_lin, emb, Wl1, bl1, Wr1, Wl2, bl2, Wr2,
           edge_index, node_ids, edge_label_index):
  src = edge_index[0].astype(jnp.int32)
  dst = edge_index[1].astype(jnp.int32)
  eli = edge_label_index.astype(jnp.int32)
  si = jnp.pad(eli[0], (0, LP - L))
  di = jnp.pad(eli[1], (0, LP - L))

  zeros_nd = jnp.zeros((N2, D), jnp.float32)

  degp = _sc_degree(dst)
  h0 = _tc_encode(x, W_lin.T, b_lin.reshape(1, D), emb)

  p1 = _sc_aggregate(h0, src, dst, zeros_nd)
  h1 = _tc_layer(p1, degp, h0, Wl1.T, Wr1.T, bl1.reshape(1, D), relu=True)

  p2 = _sc_aggregate(h1, src, dst, zeros_nd)
  h2 = _tc_layer(p2, degp, h1, Wl2.T, Wr2.T, bl2.reshape(1, D), relu=False)

  pred = _sc_classify(h2, si, di)
  return pred[:L]


# speedup vs baseline: 6.5064x; 1.3774x over previous
import dataclasses
import functools

import jax
import jax.numpy as jnp
from jax import lax
from jax.experimental import pallas as pl
from jax.experimental.pallas import tpu as pltpu
from jax.experimental.pallas import tpu_sc as plsc

N = 10000
E = 320000
L = 100000
D = 128

NC = 2
NS = 16
NW = NC * NS

N2 = 10240
RPT = N2 // NS
EC = 80
EPW = E // NW
ECHUNKS = EPW // EC

CC = 128
CPW = 25
LP = NW * CPW * CC

_mesh = functools.partial(
    plsc.VectorSubcoreMesh, core_axis_name="c", subcore_axis_name="s")


def _sc_params():
  cp = pltpu.CompilerParams()
  if "needs_layout_passes" in pltpu.CompilerParams.__dataclass_fields__:
    cp = dataclasses.replace(cp, needs_layout_passes=False)
  return cp


def _sc_degree(dst):
  @functools.partial(
      pl.kernel,
      out_type=jax.ShapeDtypeStruct((NC, N2, 16), jnp.float32),
      mesh=_mesh(),
      compiler_params=_sc_params(),
      scratch_types=[
          pltpu.VMEM((EC,), jnp.int32),
          pltpu.VMEM((80, D), jnp.float32),
          pltpu.VMEM((80,), jnp.int32),
          pltpu.VMEM((RPT // D, D), jnp.float32),
          pltpu.VMEM((RPT, 16), jnp.float32),
          pltpu.VMEM_SHARED((80, D), jnp.float32),
      ],
  )
  def k(dst_hbm, out_hbm, didx, tab, rowids, degv, rowbuf, spacc):
    c = lax.axis_index("c")
    s = lax.axis_index("s")
    wid = c * NS + s
    z16 = jnp.zeros((16,), jnp.float32)

    @pl.loop(0, 80)
    def _(i):
      for q in range(D // 16):
        tab[i, pl.ds(q * 16, 16)] = z16

    @pl.loop(0, 80, step=16)
    def _(i):
      rowids[pl.ds(i, 16)] = lax.iota(jnp.int32, 16) + i

    @pl.loop(0, RPT // D)
    def _(i):
      for q in range(D // 16):
        degv[i, pl.ds(q * 16, 16)] = z16

    pltpu.sync_copy(degv, spacc.at[pl.ds(s * (RPT // D), RPT // D)])
    plsc.subcore_barrier()

    ones = jnp.ones((16,), jnp.float32)

    @pl.loop(0, ECHUNKS)
    def _(j):
      pltpu.sync_copy(dst_hbm.at[pl.ds(wid * EPW + j * EC, EC)], didx)
      for g in range(EC // 16):
        d = didx[pl.ds(g * 16, 16)]
        plsc.addupdate_scatter(
            tab, [lax.shift_right_logical(d, 7), d & 127], ones)

    pltpu.sync_copy(tab, spacc.at[rowids], add=True)
    plsc.subcore_barrier()

    pltpu.sync_copy(spacc.at[pl.ds(s * (RPT // D), RPT // D)], degv)

    @pl.loop(0, RPT)
    def _(i):
      iv = jnp.full((16,), i, jnp.int32)
      rowbuf[i, pl.ds(0, 16)] = plsc.load_gather(
          degv, [lax.shift_right_logical(iv, 7), iv & 127])

    pltpu.sync_copy(rowbuf, out_hbm.at[c].at[pl.ds(s * RPT, RPT)])

  return k(dst)


def _sc_aggregate(h, src, dst, zeros_nd):
  @functools.partial(
      pl.kernel,
      out_type=jax.ShapeDtypeStruct((NC, N2, D), jnp.float32),
      mesh=_mesh(),
      scratch_types=[
          pltpu.VMEM((EC,), jnp.int32),
          pltpu.VMEM((EC,), jnp.int32),
          pltpu.VMEM((EC, D), jnp.float32),
          pltpu.VMEM((EC,), jnp.int32),
          pltpu.VMEM((EC,), jnp.int32),
          pltpu.VMEM((EC, D), jnp.float32),
          pltpu.VMEM_SHARED((N2, D), jnp.float32),
          pltpu.SemaphoreType.DMA,
          pltpu.SemaphoreType.DMA,
      ],
  )
  def k(h_hbm, src_hbm, dst_hbm, z_hbm, out_hbm,
        sidxA, didxA, rowsA, sidxB, didxB, rowsB, acc, semA, semB):
    c = lax.axis_index("c")
    s = lax.axis_index("s")
    wid = c * NS + s
    pltpu.sync_copy(z_hbm.at[pl.ds(s * RPT, RPT)], acc.at[pl.ds(s * RPT, RPT)])
    plsc.subcore_barrier()

    def start(j, sidx, didx, rows, sem):
      base = wid * EPW + j * EC
      pltpu.sync_copy(src_hbm.at[pl.ds(base, EC)], sidx)
      pltpu.sync_copy(dst_hbm.at[pl.ds(base, EC)], didx)
      pltpu.async_copy(h_hbm.at[sidx], rows, sem)

    def finish(sidx, didx, rows, sem):
      pltpu.make_async_copy(h_hbm.at[sidx], rows, sem).wait()
      pltpu.sync_copy(rows, acc.at[didx], add=True)

    start(0, sidxA, didxA, rowsA, semA)

    @pl.loop(0, ECHUNKS - 1, step=2)
    def _(j):
      start(j + 1, sidxB, didxB, rowsB, semB)
      finish(sidxA, didxA, rowsA, semA)
      start(j + 2, sidxA, didxA, rowsA, semA)
      finish(sidxB, didxB, rowsB, semB)

    finish(sidxA, didxA, rowsA, semA)
    plsc.subcore_barrier()
    pltpu.sync_copy(acc.at[pl.ds(s * RPT, RPT)],
                    out_hbm.at[c].at[pl.ds(s * RPT, RPT)])

  return k(h, src, dst, zeros_nd)


def _sc_classify(h, si, di):
  @functools.partial(
      pl.kernel,
      out_type=jax.ShapeDtypeStruct((LP,), jnp.float32),
      mesh=_mesh(),
      compiler_params=_sc_params(),
      scratch_types=[
          pltpu.VMEM((CC,), jnp.int32),
          pltpu.VMEM((CC,), jnp.int32),
          pltpu.VMEM((CC, D), jnp.float32),
          pltpu.VMEM((CC, D), jnp.float32),
          pltpu.VMEM((CC,), jnp.int32),
          pltpu.VMEM((CC,), jnp.int32),
          pltpu.VMEM((CC, D), jnp.float32),
          pltpu.VMEM((CC, D), jnp.float32),
          pltpu.VMEM((CC,), jnp.float32),
          pltpu.VMEM((16 * 17,), jnp.float32),
          pltpu.SemaphoreType.DMA,
          pltpu.SemaphoreType.DMA,
      ],
  )
  def k(h_hbm, si_hbm, di_hbm, out_hbm,
        sidxA, didxA, arowsA, browsA,
        sidxB, didxB, arowsB, browsB,
        ovec, tile17, semA, semB):
    c = lax.axis_index("c")
    s = lax.axis_index("s")
    wid = c * NS + s
    lanes = lax.iota(jnp.int32, 16)

    def start(j, sidx, didx, arows, brows, sem):
      base = (wid * CPW + j) * CC
      pltpu.sync_copy(si_hbm.at[pl.ds(base, CC)], sidx)
      pltpu.sync_copy(di_hbm.at[pl.ds(base, CC)], didx)
      pltpu.async_copy(h_hbm.at[sidx], arows, sem)
      pltpu.async_copy(h_hbm.at[didx], brows, sem)

    def wait(sidx, didx, arows, brows, sem):
      pltpu.make_async_copy(h_hbm.at[sidx], arows, sem).wait()
      pltpu.make_async_copy(h_hbm.at[didx], brows, sem).wait()

    def compute(j, arows, brows):
      @pl.loop(0, CC, step=16)
      def _(p0):
        for p in range(16):
          pi = p0 + p
          acc = arows[pi, pl.ds(0, 16)] * brows[pi, pl.ds(0, 16)]
          for q in range(1, D // 16):
            acc += arows[pi, pl.ds(q * 16, 16)] * brows[pi, pl.ds(q * 16, 16)]
          plsc.store_scatter(tile17, [lanes + 17 * p], acc)
        sums = plsc.load_gather(tile17, [lanes * 17])
        for kcol in range(1, 16):
          sums += plsc.load_gather(tile17, [lanes * 17 + kcol])
        ovec[pl.ds(p0, 16)] = sums

      pltpu.sync_copy(ovec, out_hbm.at[pl.ds((wid * CPW + j) * CC, CC)])

    start(0, sidxA, didxA, arowsA, browsA, semA)

    @pl.loop(0, CPW - 1, step=2)
    def _(j):
      start(j + 1, sidxB, didxB, arowsB, browsB, semB)
      wait(sidxA, didxA, arowsA, browsA, semA)
      compute(j, arowsA, browsA)
      start(j + 2, sidxA, didxA, arowsA, browsA, semA)
      wait(sidxB, didxB, arowsB, browsB, semB)
      compute(j + 1, arowsB, browsB)

    wait(sidxA, didxA, arowsA, browsA, semA)
    compute(CPW - 1, arowsA, browsA)

  return k(h, si, di)


def _tc_encode(x, w_t, b, emb):
  BM = 1000

  def body(x_ref, w_ref, b_ref, e_ref, o_ref):
    o_ref[...] = (
        jax.lax.dot(x_ref[...], w_ref[...],
                    precision=lax.Precision.HIGHEST,
                    preferred_element_type=jnp.float32)
        + b_ref[...] + e_ref[...])

  return pl.pallas_call(
      body,
      grid=(N // BM,),
      in_specs=[
          pl.BlockSpec((BM, D), lambda i: (i, 0)),
          pl.BlockSpec((D, D), lambda i: (0, 0)),
          pl.BlockSpec((1, D), lambda i: (0, 0)),
          pl.BlockSpec((BM, D), lambda i: (i, 0)),
      ],
      out_specs=pl.BlockSpec((BM, D), lambda i: (i, 0)),
      out_shape=jax.ShapeDtypeStruct((N, D), jnp.float32),
  )(x, w_t, b, emb)


def _tc_layer(p, degp, h_prev, wl_t, wr_t, bl, relu):
  BM = 1000

  def body(p_ref, d_ref, h_ref, wl_ref, wr_ref, b_ref, o_ref):
    agg = p_ref[0] + p_ref[1]
    deg = d_ref[0, :, 0:1] + d_ref[1, :, 0:1]
    mean = agg / jnp.maximum(deg, 1.0)
    out = (
        jax.lax.dot(mean, wl_ref[...], precision=lax.Precision.HIGHEST,
                    preferred_element_type=jnp.float32)
        + jax.lax.dot(h_ref[...], wr_ref[...],
                      precision=lax.Precision.HIGHEST,
                      preferred_element_type=jnp.float32)
        + b_ref[...])
    if relu:
      out = jnp.maximum(out, 0.0)
    o_ref[...] = out

  return pl.pallas_call(
      body,
      grid=(N // BM,),
      in_specs=[
          pl.BlockSpec((NC, BM, D), lambda i: (0, i, 0)),
          pl.BlockSpec((NC, BM, 16), lambda i: (0, i, 0)),
          pl.BlockSpec((BM, D), lambda i: (i, 0)),
          pl.BlockSpec((D, D), lambda i: (0, 0)),
          pl.BlockSpec((D, D), lambda i: (0, 0)),
          pl.BlockSpec((1, D), lambda i: (0, 0)),
      ],
      out_specs=pl.BlockSpec((BM, D), lambda i: (i, 0)),
      out_shape=jax.ShapeDtypeStruct((N, D), jnp.float32),
  )(p, degp, h_prev, wl_t, wr_t, bl)


def kernel(x, W_lin, b_lin, emb, Wl1, bl1, Wr1, Wl2, bl2, Wr2,
           edge_index, node_ids, edge_label_index):
  src = edge_index[0].astype(jnp.int32)
  dst = edge_index[1].astype(jnp.int32)
  eli = edge_label_index.astype(jnp.int32)
  padv = (jnp.arange(LP - L, dtype=jnp.int32) * 97) % N
  si = jnp.concatenate([eli[0], padv])
  di = jnp.concatenate([eli[1], padv])

  zeros_nd = jnp.zeros((N2, D), jnp.float32)

  degp = _sc_degree(dst)
  h0 = _tc_encode(x, W_lin.T, b_lin.reshape(1, D), emb)

  p1 = _sc_aggregate(h0, src, dst, zeros_nd)
  h1 = _tc_layer(p1, degp, h0, Wl1.T, Wr1.T, bl1.reshape(1, D), relu=True)

  p2 = _sc_aggregate(h1, src, dst, zeros_nd)
  h2 = _tc_layer(p2, degp, h1, Wl2.T, Wr2.T, bl2.reshape(1, D), relu=False)

  pred = _sc_classify(h2, si, di)
  return pred[:L]


# speedup vs baseline: 6.5180x; 1.0018x over previous
import dataclasses
import functools

import jax
import jax.numpy as jnp
from jax import lax
from jax.experimental import pallas as pl
from jax.experimental.pallas import tpu as pltpu
from jax.experimental.pallas import tpu_sc as plsc

N = 10000
E = 320000
L = 100000
D = 128

NC = 2
NS = 16
NW = NC * NS

N2 = 10240
RPT = N2 // NS
EC = 80
EPW = E // NW
ECHUNKS = EPW // EC

CC = 128
CPW = 25
LP = NW * CPW * CC

_mesh = functools.partial(
    plsc.VectorSubcoreMesh, core_axis_name="c", subcore_axis_name="s")


def _sc_params():
  cp = pltpu.CompilerParams()
  if "needs_layout_passes" in pltpu.CompilerParams.__dataclass_fields__:
    cp = dataclasses.replace(cp, needs_layout_passes=False)
  return cp


def _sc_degree(dst):
  @functools.partial(
      pl.kernel,
      out_type=jax.ShapeDtypeStruct((NC, N2, 16), jnp.float32),
      mesh=_mesh(),
      compiler_params=_sc_params(),
      scratch_types=[
          pltpu.VMEM((EC,), jnp.int32),
          pltpu.VMEM((80, D), jnp.float32),
          pltpu.VMEM((80,), jnp.int32),
          pltpu.VMEM((RPT // D, D), jnp.float32),
          pltpu.VMEM((RPT, 16), jnp.float32),
          pltpu.VMEM_SHARED((80, D), jnp.float32),
      ],
  )
  def k(dst_hbm, out_hbm, didx, tab, rowids, degv, rowbuf, spacc):
    c = lax.axis_index("c")
    s = lax.axis_index("s")
    wid = c * NS + s
    z16 = jnp.zeros((16,), jnp.float32)

    @pl.loop(0, 80)
    def _(i):
      for q in range(D // 16):
        tab[i, pl.ds(q * 16, 16)] = z16

    @pl.loop(0, 80, step=16)
    def _(i):
      rowids[pl.ds(i, 16)] = lax.iota(jnp.int32, 16) + i

    @pl.loop(0, RPT // D)
    def _(i):
      for q in range(D // 16):
        degv[i, pl.ds(q * 16, 16)] = z16

    pltpu.sync_copy(degv, spacc.at[pl.ds(s * (RPT // D), RPT // D)])
    plsc.subcore_barrier()

    ones = jnp.ones((16,), jnp.float32)

    @pl.loop(0, ECHUNKS)
    def _(j):
      pltpu.sync_copy(dst_hbm.at[pl.ds(wid * EPW + j * EC, EC)], didx)
      for g in range(EC // 16):
        d = didx[pl.ds(g * 16, 16)]
        plsc.addupdate_scatter(
            tab, [lax.shift_right_logical(d, 7), d & 127], ones)

    pltpu.sync_copy(tab, spacc.at[rowids], add=True)
    plsc.subcore_barrier()

    pltpu.sync_copy(spacc.at[pl.ds(s * (RPT // D), RPT // D)], degv)

    @pl.loop(0, RPT)
    def _(i):
      iv = jnp.full((16,), i, jnp.int32)
      rowbuf[i, pl.ds(0, 16)] = plsc.load_gather(
          degv, [lax.shift_right_logical(iv, 7), iv & 127])

    pltpu.sync_copy(rowbuf, out_hbm.at[c].at[pl.ds(s * RPT, RPT)])

  return k(dst)


def _sc_aggregate(h, src, dst, zeros_nd):
  @functools.partial(
      pl.kernel,
      out_type=jax.ShapeDtypeStruct((NC, N2, D), jnp.float32),
      mesh=_mesh(),
      scratch_types=[
          pltpu.VMEM((EC,), jnp.int32),
          pltpu.VMEM((EC,), jnp.int32),
          pltpu.VMEM((EC, D), jnp.float32),
          pltpu.VMEM((EC,), jnp.int32),
          pltpu.VMEM((EC,), jnp.int32),
          pltpu.VMEM((EC, D), jnp.float32),
          pltpu.VMEM_SHARED((N2, D), jnp.float32),
          pltpu.SemaphoreType.DMA,
          pltpu.SemaphoreType.DMA,
      ],
  )
  def k(h_hbm, src_hbm, dst_hbm, z_hbm, out_hbm,
        sidxA, didxA, rowsA, sidxB, didxB, rowsB, acc, semA, semB):
    c = lax.axis_index("c")
    s = lax.axis_index("s")
    wid = c * NS + s

    def start(j, sidx, didx, rows, sem):
      base = wid * EPW + j * EC
      pltpu.sync_copy(src_hbm.at[pl.ds(base, EC)], sidx)
      pltpu.sync_copy(dst_hbm.at[pl.ds(base, EC)], didx)
      pltpu.async_copy(h_hbm.at[sidx], rows, sem)

    def finish(sidx, didx, rows, sem):
      pltpu.make_async_copy(h_hbm.at[sidx], rows, sem).wait()
      pltpu.sync_copy(rows, acc.at[didx], add=True)

    start(0, sidxA, didxA, rowsA, semA)
    pltpu.sync_copy(z_hbm.at[pl.ds(s * RPT, RPT)], acc.at[pl.ds(s * RPT, RPT)])
    plsc.subcore_barrier()

    @pl.loop(0, ECHUNKS - 1, step=2)
    def _(j):
      start(j + 1, sidxB, didxB, rowsB, semB)
      finish(sidxA, didxA, rowsA, semA)
      start(j + 2, sidxA, didxA, rowsA, semA)
      finish(sidxB, didxB, rowsB, semB)

    finish(sidxA, didxA, rowsA, semA)
    plsc.subcore_barrier()
    pltpu.sync_copy(acc.at[pl.ds(s * RPT, RPT)],
                    out_hbm.at[c].at[pl.ds(s * RPT, RPT)])

  return k(h, src, dst, zeros_nd)


def _sc_classify(h, si, di):
  @functools.partial(
      pl.kernel,
      out_type=jax.ShapeDtypeStruct((LP,), jnp.float32),
      mesh=_mesh(),
      compiler_params=_sc_params(),
      scratch_types=[
          pltpu.VMEM((CC,), jnp.int32),
          pltpu.VMEM((CC,), jnp.int32),
          pltpu.VMEM((CC, D), jnp.float32),
          pltpu.VMEM((CC, D), jnp.float32),
          pltpu.VMEM((CC,), jnp.int32),
          pltpu.VMEM((CC,), jnp.int32),
          pltpu.VMEM((CC, D), jnp.float32),
          pltpu.VMEM((CC, D), jnp.float32),
          pltpu.VMEM((CC,), jnp.float32),
          pltpu.VMEM((16 * 17,), jnp.float32),
          pltpu.SemaphoreType.DMA,
          pltpu.SemaphoreType.DMA,
      ],
  )
  def k(h_hbm, si_hbm, di_hbm, out_hbm,
        sidxA, didxA, arowsA, browsA,
        sidxB, didxB, arowsB, browsB,
        ovec, tile17, semA, semB):
    c = lax.axis_index("c")
    s = lax.axis_index("s")
    wid = c * NS + s
    lanes = lax.iota(jnp.int32, 16)

    def start(j, sidx, didx, arows, brows, sem):
      base = (wid * CPW + j) * CC
      pltpu.sync_copy(si_hbm.at[pl.ds(base, CC)], sidx)
      pltpu.sync_copy(di_hbm.at[pl.ds(base, CC)], didx)
      pltpu.async_copy(h_hbm.at[sidx], arows, sem)
      pltpu.async_copy(h_hbm.at[didx], brows, sem)

    def wait(sidx, didx, arows, brows, sem):
      pltpu.make_async_copy(h_hbm.at[sidx], arows, sem).wait()
      pltpu.make_async_copy(h_hbm.at[didx], brows, sem).wait()

    def compute(j, arows, brows):
      @pl.loop(0, CC, step=16)
      def _(p0):
        for p in range(16):
          pi = p0 + p
          acc = arows[pi, pl.ds(0, 16)] * brows[pi, pl.ds(0, 16)]
          for q in range(1, D // 16):
            acc += arows[pi, pl.ds(q * 16, 16)] * brows[pi, pl.ds(q * 16, 16)]
          plsc.store_scatter(tile17, [lanes + 17 * p], acc)
        sums = plsc.load_gather(tile17, [lanes * 17])
        for kcol in range(1, 16):
          sums += plsc.load_gather(tile17, [lanes * 17 + kcol])
        ovec[pl.ds(p0, 16)] = sums

      pltpu.sync_copy(ovec, out_hbm.at[pl.ds((wid * CPW + j) * CC, CC)])

    start(0, sidxA, didxA, arowsA, browsA, semA)

    @pl.loop(0, CPW - 1, step=2)
    def _(j):
      start(j + 1, sidxB, didxB, arowsB, browsB, semB)
      wait(sidxA, didxA, arowsA, browsA, semA)
      compute(j, arowsA, browsA)
      start(j + 2, sidxA, didxA, arowsA, browsA, semA)
      wait(sidxB, didxB, arowsB, browsB, semB)
      compute(j + 1, arowsB, browsB)

    wait(sidxA, didxA, arowsA, browsA, semA)
    compute(CPW - 1, arowsA, browsA)

  return k(h, si, di)


def _tc_encode(x, w_t, b, emb):
  BM = 1000

  def body(x_ref, w_ref, b_ref, e_ref, o_ref):
    o_ref[...] = (
        jax.lax.dot(x_ref[...], w_ref[...],
                    precision=lax.Precision.HIGHEST,
                    preferred_element_type=jnp.float32)
        + b_ref[...] + e_ref[...])

  return pl.pallas_call(
      body,
      grid=(N // BM,),
      in_specs=[
          pl.BlockSpec((BM, D), lambda i: (i, 0)),
          pl.BlockSpec((D, D), lambda i: (0, 0)),
          pl.BlockSpec((1, D), lambda i: (0, 0)),
          pl.BlockSpec((BM, D), lambda i: (i, 0)),
      ],
      out_specs=pl.BlockSpec((BM, D), lambda i: (i, 0)),
      out_shape=jax.ShapeDtypeStruct((N, D), jnp.float32),
  )(x, w_t, b, emb)


def _tc_layer(p, degp, h_prev, wl_t, wr_t, bl, relu):
  BM = 1000

  def body(p_ref, d_ref, h_ref, wl_ref, wr_ref, b_ref, o_ref):
    agg = p_ref[0] + p_ref[1]
    deg = d_ref[0, :, 0:1] + d_ref[1, :, 0:1]
    mean = agg / jnp.maximum(deg, 1.0)
    out = (
        jax.lax.dot(mean, wl_ref[...], precision=lax.Precision.HIGHEST,
                    preferred_element_type=jnp.float32)
        + jax.lax.dot(h_ref[...], wr_ref[...],
                      precision=lax.Precision.HIGHEST,
                      preferred_element_type=jnp.float32)
        + b_ref[...])
    if relu:
      out = jnp.maximum(out, 0.0)
    o_ref[...] = out

  return pl.pallas_call(
      body,
      grid=(N // BM,),
      in_specs=[
          pl.BlockSpec((NC, BM, D), lambda i: (0, i, 0)),
          pl.BlockSpec((NC, BM, 16), lambda i: (0, i, 0)),
          pl.BlockSpec((BM, D), lambda i: (i, 0)),
          pl.BlockSpec((D, D), lambda i: (0, 0)),
          pl.BlockSpec((D, D), lambda i: (0, 0)),
          pl.BlockSpec((1, D), lambda i: (0, 0)),
      ],
      out_specs=pl.BlockSpec((BM, D), lambda i: (i, 0)),
      out_shape=jax.ShapeDtypeStruct((N, D), jnp.float32),
  )(p, degp, h_prev, wl_t, wr_t, bl)


def kernel(x, W_lin, b_lin, emb, Wl1, bl1, Wr1, Wl2, bl2, Wr2,
           edge_index, node_ids, edge_label_index):
  src = edge_index[0].astype(jnp.int32)
  dst = edge_index[1].astype(jnp.int32)
  eli = edge_label_index.astype(jnp.int32)
  padv = (jnp.arange(LP - L, dtype=jnp.int32) * 97) % N
  si = jnp.concatenate([eli[0], padv])
  di = jnp.concatenate([eli[1], padv])

  zeros_nd = jnp.zeros((N2, D), jnp.float32)

  degp = _sc_degree(dst)
  h0 = _tc_encode(x, W_lin.T, b_lin.reshape(1, D), emb)

  p1 = _sc_aggregate(h0, src, dst, zeros_nd)
  h1 = _tc_layer(p1, degp, h0, Wl1.T, Wr1.T, bl1.reshape(1, D), relu=True)

  p2 = _sc_aggregate(h1, src, dst, zeros_nd)
  h2 = _tc_layer(p2, degp, h1, Wl2.T, Wr2.T, bl2.reshape(1, D), relu=False)

  pred = _sc_classify(h2, si, di)
  return pred[:L]


# speedup vs baseline: 7.1323x; 1.0942x over previous
import dataclasses
import functools

import jax
import jax.numpy as jnp
from jax import lax
from jax.experimental import pallas as pl
from jax.experimental.pallas import tpu as pltpu
from jax.experimental.pallas import tpu_sc as plsc

N = 10000
E = 320000
L = 100000
D = 128

NC = 2
NS = 16
NW = NC * NS

N2 = 10240
RPT = N2 // NS
EC = 80
EPW = E // NW
ECHUNKS = EPW // EC

CC = 128
CPW = 25
LP = NW * CPW * CC

_mesh = functools.partial(
    plsc.VectorSubcoreMesh, core_axis_name="c", subcore_axis_name="s")


def _sc_params():
  cp = pltpu.CompilerParams()
  if "needs_layout_passes" in pltpu.CompilerParams.__dataclass_fields__:
    cp = dataclasses.replace(cp, needs_layout_passes=False)
  return cp


def _sc_degree(dst):
  @functools.partial(
      pl.kernel,
      out_type=jax.ShapeDtypeStruct((NC, N2, 16), jnp.float32),
      mesh=_mesh(),
      compiler_params=_sc_params(),
      scratch_types=[
          pltpu.VMEM((EPW,), jnp.int32),
          pltpu.VMEM((80, D), jnp.float32),
          pltpu.VMEM((80,), jnp.int32),
          pltpu.VMEM((RPT // D, D), jnp.float32),
          pltpu.VMEM((RPT, 16), jnp.float32),
          pltpu.VMEM_SHARED((80, D), jnp.float32),
      ],
  )
  def k(dst_hbm, out_hbm, didx, tab, rowids, degv, rowbuf, spacc):
    c = lax.axis_index("c")
    s = lax.axis_index("s")
    wid = c * NS + s
    z16 = jnp.zeros((16,), jnp.float32)
    pltpu.sync_copy(dst_hbm.at[pl.ds(wid * EPW, EPW)], didx)

    @pl.loop(0, 80)
    def _(i):
      for q in range(D // 16):
        tab[i, pl.ds(q * 16, 16)] = z16

    @pl.loop(0, 80, step=16)
    def _(i):
      rowids[pl.ds(i, 16)] = lax.iota(jnp.int32, 16) + i

    @pl.loop(0, RPT // D)
    def _(i):
      for q in range(D // 16):
        degv[i, pl.ds(q * 16, 16)] = z16

    pltpu.sync_copy(degv, spacc.at[pl.ds(s * (RPT // D), RPT // D)])
    plsc.subcore_barrier()

    ones = jnp.ones((16,), jnp.float32)

    @pl.loop(0, EPW, step=16 * 5)
    def _(j):
      for g in range(5):
        d = didx[pl.ds(j + g * 16, 16)]
        plsc.addupdate_scatter(
            tab, [lax.shift_right_logical(d, 7), d & 127], ones)

    pltpu.sync_copy(tab, spacc.at[rowids], add=True)
    plsc.subcore_barrier()

    pltpu.sync_copy(spacc.at[pl.ds(s * (RPT // D), RPT // D)], degv)

    @pl.loop(0, RPT)
    def _(i):
      iv = jnp.full((16,), i, jnp.int32)
      rowbuf[i, pl.ds(0, 16)] = plsc.load_gather(
          degv, [lax.shift_right_logical(iv, 7), iv & 127])

    pltpu.sync_copy(rowbuf, out_hbm.at[c].at[pl.ds(s * RPT, RPT)])

  return k(dst)


def _sc_aggregate(h, src, dst, zeros_nd):
  @functools.partial(
      pl.kernel,
      out_type=jax.ShapeDtypeStruct((NC, N2, D), jnp.float32),
      mesh=_mesh(),
      scratch_types=[
          pltpu.VMEM((EC,), jnp.int32),
          pltpu.VMEM((EC,), jnp.int32),
          pltpu.VMEM((EC, D), jnp.float32),
          pltpu.VMEM((EC,), jnp.int32),
          pltpu.VMEM((EC,), jnp.int32),
          pltpu.VMEM((EC, D), jnp.float32),
          pltpu.VMEM_SHARED((N2, D), jnp.float32),
          pltpu.SemaphoreType.DMA,
          pltpu.SemaphoreType.DMA,
      ],
  )
  def k(h_hbm, src_hbm, dst_hbm, z_hbm, out_hbm,
        sidxA, didxA, rowsA, sidxB, didxB, rowsB, acc, semA, semB):
    c = lax.axis_index("c")
    s = lax.axis_index("s")
    wid = c * NS + s

    def start(j, sidx, didx, rows, sem):
      base = wid * EPW + j * EC
      pltpu.sync_copy(src_hbm.at[pl.ds(base, EC)], sidx)
      pltpu.sync_copy(dst_hbm.at[pl.ds(base, EC)], didx)
      pltpu.async_copy(h_hbm.at[sidx], rows, sem)

    def finish(sidx, didx, rows, sem):
      pltpu.make_async_copy(h_hbm.at[sidx], rows, sem).wait()
      pltpu.sync_copy(rows, acc.at[didx], add=True)

    start(0, sidxA, didxA, rowsA, semA)
    pltpu.sync_copy(z_hbm.at[pl.ds(s * RPT, RPT)], acc.at[pl.ds(s * RPT, RPT)])
    plsc.subcore_barrier()

    @pl.loop(0, ECHUNKS - 1, step=2)
    def _(j):
      start(j + 1, sidxB, didxB, rowsB, semB)
      finish(sidxA, didxA, rowsA, semA)
      start(j + 2, sidxA, didxA, rowsA, semA)
      finish(sidxB, didxB, rowsB, semB)

    finish(sidxA, didxA, rowsA, semA)
    plsc.subcore_barrier()
    pltpu.sync_copy(acc.at[pl.ds(s * RPT, RPT)],
                    out_hbm.at[c].at[pl.ds(s * RPT, RPT)])

  return k(h, src, dst, zeros_nd)


def _sc_classify(h, si, di):
  @functools.partial(
      pl.kernel,
      out_type=jax.ShapeDtypeStruct((LP,), jnp.float32),
      mesh=_mesh(),
      compiler_params=_sc_params(),
      scratch_types=[
          pltpu.VMEM((CC,), jnp.int32),
          pltpu.VMEM((CC,), jnp.int32),
          pltpu.VMEM((CC, D), jnp.float32),
          pltpu.VMEM((CC, D), jnp.float32),
          pltpu.VMEM((CC,), jnp.int32),
          pltpu.VMEM((CC,), jnp.int32),
          pltpu.VMEM((CC, D), jnp.float32),
          pltpu.VMEM((CC, D), jnp.float32),
          pltpu.VMEM((CC,), jnp.float32),
          pltpu.VMEM((16 * 17,), jnp.float32),
          pltpu.SemaphoreType.DMA,
          pltpu.SemaphoreType.DMA,
      ],
  )
  def k(h_hbm, si_hbm, di_hbm, out_hbm,
        sidxA, didxA, arowsA, browsA,
        sidxB, didxB, arowsB, browsB,
        ovec, tile17, semA, semB):
    c = lax.axis_index("c")
    s = lax.axis_index("s")
    wid = c * NS + s
    lanes = lax.iota(jnp.int32, 16)

    def start(j, sidx, didx, arows, brows, sem):
      base = (wid * CPW + j) * CC
      pltpu.sync_copy(si_hbm.at[pl.ds(base, CC)], sidx)
      pltpu.sync_copy(di_hbm.at[pl.ds(base, CC)], didx)
      pltpu.async_copy(h_hbm.at[sidx], arows, sem)
      pltpu.async_copy(h_hbm.at[didx], brows, sem)

    def wait(sidx, didx, arows, brows, sem):
      pltpu.make_async_copy(h_hbm.at[sidx], arows, sem).wait()
      pltpu.make_async_copy(h_hbm.at[didx], brows, sem).wait()

    def compute(j, arows, brows):
      @pl.loop(0, CC, step=16)
      def _(p0):
        for p in range(16):
          pi = p0 + p
          acc = arows[pi, pl.ds(0, 16)] * brows[pi, pl.ds(0, 16)]
          for q in range(1, D // 16):
            acc += arows[pi, pl.ds(q * 16, 16)] * brows[pi, pl.ds(q * 16, 16)]
          plsc.store_scatter(tile17, [lanes + 17 * p], acc)
        sums = plsc.load_gather(tile17, [lanes * 17])
        for kcol in range(1, 16):
          sums += plsc.load_gather(tile17, [lanes * 17 + kcol])
        ovec[pl.ds(p0, 16)] = sums

      pltpu.sync_copy(ovec, out_hbm.at[pl.ds((wid * CPW + j) * CC, CC)])

    start(0, sidxA, didxA, arowsA, browsA, semA)

    @pl.loop(0, CPW - 1, step=2)
    def _(j):
      start(j + 1, sidxB, didxB, arowsB, browsB, semB)
      wait(sidxA, didxA, arowsA, browsA, semA)
      compute(j, arowsA, browsA)
      start(j + 2, sidxA, didxA, arowsA, browsA, semA)
      wait(sidxB, didxB, arowsB, browsB, semB)
      compute(j + 1, arowsB, browsB)

    wait(sidxA, didxA, arowsA, browsA, semA)
    compute(CPW - 1, arowsA, browsA)

  return k(h, si, di)


def _tc_encode(x, w_t, b, emb):
  BM = 1000

  def body(x_ref, w_ref, b_ref, e_ref, o_ref):
    o_ref[...] = (
        jax.lax.dot(x_ref[...], w_ref[...],
                    precision=lax.Precision.HIGHEST,
                    preferred_element_type=jnp.float32)
        + b_ref[...] + e_ref[...])

  return pl.pallas_call(
      body,
      grid=(N // BM,),
      in_specs=[
          pl.BlockSpec((BM, D), lambda i: (i, 0)),
          pl.BlockSpec((D, D), lambda i: (0, 0)),
          pl.BlockSpec((1, D), lambda i: (0, 0)),
          pl.BlockSpec((BM, D), lambda i: (i, 0)),
      ],
      out_specs=pl.BlockSpec((BM, D), lambda i: (i, 0)),
      out_shape=jax.ShapeDtypeStruct((N, D), jnp.float32),
  )(x, w_t, b, emb)


def _tc_layer(p, degp, h_prev, wl_t, wr_t, bl, relu):
  BM = 1000

  def body(p_ref, d_ref, h_ref, wl_ref, wr_ref, b_ref, o_ref):
    agg = p_ref[0] + p_ref[1]
    deg = d_ref[0, :, 0:1] + d_ref[1, :, 0:1]
    mean = agg / jnp.maximum(deg, 1.0)
    out = (
        jax.lax.dot(mean, wl_ref[...], precision=lax.Precision.HIGHEST,
                    preferred_element_type=jnp.float32)
        + jax.lax.dot(h_ref[...], wr_ref[...],
                      precision=lax.Precision.HIGHEST,
                      preferred_element_type=jnp.float32)
        + b_ref[...])
    if relu:
      out = jnp.maximum(out, 0.0)
    o_ref[...] = out

  return pl.pallas_call(
      body,
      grid=(N // BM,),
      in_specs=[
          pl.BlockSpec((NC, BM, D), lambda i: (0, i, 0)),
          pl.BlockSpec((NC, BM, 16), lambda i: (0, i, 0)),
          pl.BlockSpec((BM, D), lambda i: (i, 0)),
          pl.BlockSpec((D, D), lambda i: (0, 0)),
          pl.BlockSpec((D, D), lambda i: (0, 0)),
          pl.BlockSpec((1, D), lambda i: (0, 0)),
      ],
      out_specs=pl.BlockSpec((BM, D), lambda i: (i, 0)),
      out_shape=jax.ShapeDtypeStruct((N, D), jnp.float32),
  )(p, degp, h_prev, wl_t, wr_t, bl)


def kernel(x, W_lin, b_lin, emb, Wl1, bl1, Wr1, Wl2, bl2, Wr2,
           edge_index, node_ids, edge_label_index):
  src = edge_index[0].astype(jnp.int32)
  dst = edge_index[1].astype(jnp.int32)
  eli = edge_label_index.astype(jnp.int32)
  padv = (jnp.arange(LP - L, dtype=jnp.int32) * 97) % N
  si = jnp.concatenate([eli[0], padv])
  di = jnp.concatenate([eli[1], padv])

  zeros_nd = jnp.zeros((N2, D), jnp.float32)

  degp = _sc_degree(dst)
  h0 = _tc_encode(x, W_lin.T, b_lin.reshape(1, D), emb)

  p1 = _sc_aggregate(h0, src, dst, zeros_nd)
  h1 = _tc_layer(p1, degp, h0, Wl1.T, Wr1.T, bl1.reshape(1, D), relu=True)

  p2 = _sc_aggregate(h1, src, dst, zeros_nd)
  h2 = _tc_layer(p2, degp, h1, Wl2.T, Wr2.T, bl2.reshape(1, D), relu=False)

  pred = _sc_classify(h2, si, di)
  return pred[:L]


# speedup vs baseline: 8.1465x; 1.1422x over previous
import dataclasses
import functools

import jax
import jax.numpy as jnp
from jax import lax
from jax.experimental import pallas as pl
from jax.experimental.pallas import tpu as pltpu
from jax.experimental.pallas import tpu_sc as plsc

N = 10000
E = 320000
L = 100000
D = 128

NC = 2
NS = 16
NW = NC * NS

N2 = 10240
RPT = N2 // NS
EC = 128
ECHUNKS = 79
EPW = EC * ECHUNKS
E2 = EPW * NW

CC = 128
CPW = 25
LP = NW * CPW * CC

_mesh = functools.partial(
    plsc.VectorSubcoreMesh, core_axis_name="c", subcore_axis_name="s")


def _sc_params():
  cp = pltpu.CompilerParams()
  if "needs_layout_passes" in pltpu.CompilerParams.__dataclass_fields__:
    cp = dataclasses.replace(cp, needs_layout_passes=False)
  return cp


def _sc_degree(dst):
  @functools.partial(
      pl.kernel,
      out_type=jax.ShapeDtypeStruct((NC, N2, 16), jnp.float32),
      mesh=_mesh(),
      compiler_params=_sc_params(),
      scratch_types=[
          pltpu.VMEM((EPW,), jnp.int32),
          pltpu.VMEM((80, D), jnp.float32),
          pltpu.VMEM((80,), jnp.int32),
          pltpu.VMEM((RPT // D, D), jnp.float32),
          pltpu.VMEM((RPT, 16), jnp.float32),
          pltpu.VMEM_SHARED((80, D), jnp.float32),
      ],
  )
  def k(dst_hbm, out_hbm, didx, tab, rowids, degv, rowbuf, spacc):
    c = lax.axis_index("c")
    s = lax.axis_index("s")
    wid = c * NS + s
    z16 = jnp.zeros((16,), jnp.float32)
    pltpu.sync_copy(dst_hbm.at[pl.ds(wid * EPW, EPW)], didx)

    @pl.loop(0, 80)
    def _(i):
      for q in range(D // 16):
        tab[i, pl.ds(q * 16, 16)] = z16

    @pl.loop(0, 80, step=16)
    def _(i):
      rowids[pl.ds(i, 16)] = lax.iota(jnp.int32, 16) + i

    @pl.loop(0, RPT // D)
    def _(i):
      for q in range(D // 16):
        degv[i, pl.ds(q * 16, 16)] = z16

    pltpu.sync_copy(degv, spacc.at[pl.ds(s * (RPT // D), RPT // D)])
    plsc.subcore_barrier()

    ones = jnp.ones((16,), jnp.float32)

    @pl.loop(0, EPW, step=16 * 4)
    def _(j):
      for g in range(4):
        d = didx[pl.ds(j + g * 16, 16)]
        plsc.addupdate_scatter(
            tab, [lax.shift_right_logical(d, 7), d & 127], ones)

    pltpu.sync_copy(tab, spacc.at[rowids], add=True)
    plsc.subcore_barrier()

    pltpu.sync_copy(spacc.at[pl.ds(s * (RPT // D), RPT // D)], degv)

    @pl.loop(0, RPT)
    def _(i):
      iv = jnp.full((16,), i, jnp.int32)
      rowbuf[i, pl.ds(0, 16)] = plsc.load_gather(
          degv, [lax.shift_right_logical(iv, 7), iv & 127])

    pltpu.sync_copy(rowbuf, out_hbm.at[c].at[pl.ds(s * RPT, RPT)])

  return k(dst)


def _sc_aggregate(h, src, dst, zeros_nd):
  @functools.partial(
      pl.kernel,
      out_type=jax.ShapeDtypeStruct((NC, N2, D), jnp.float32),
      mesh=_mesh(),
      scratch_types=[
          pltpu.VMEM((EC,), jnp.int32),
          pltpu.VMEM((EC,), jnp.int32),
          pltpu.VMEM((EC, D), jnp.float32),
          pltpu.VMEM((EC,), jnp.int32),
          pltpu.VMEM((EC,), jnp.int32),
          pltpu.VMEM((EC, D), jnp.float32),
          pltpu.VMEM_SHARED((N2, D), jnp.float32),
          pltpu.SemaphoreType.DMA,
          pltpu.SemaphoreType.DMA,
      ],
  )
  def k(h_hbm, src_hbm, dst_hbm, z_hbm, out_hbm,
        sidxA, didxA, rowsA, sidxB, didxB, rowsB, acc, semA, semB):
    c = lax.axis_index("c")
    s = lax.axis_index("s")
    wid = c * NS + s

    def start(j, sidx, didx, rows, sem):
      base = wid * EPW + j * EC
      pltpu.sync_copy(src_hbm.at[pl.ds(base, EC)], sidx)
      pltpu.sync_copy(dst_hbm.at[pl.ds(base, EC)], didx)
      pltpu.async_copy(h_hbm.at[sidx], rows, sem)

    def finish(sidx, didx, rows, sem):
      pltpu.make_async_copy(h_hbm.at[sidx], rows, sem).wait()
      pltpu.sync_copy(rows, acc.at[didx], add=True)

    start(0, sidxA, didxA, rowsA, semA)
    pltpu.sync_copy(z_hbm.at[pl.ds(s * RPT, RPT)], acc.at[pl.ds(s * RPT, RPT)])
    plsc.subcore_barrier()

    @pl.loop(0, ECHUNKS - 1, step=2)
    def _(j):
      start(j + 1, sidxB, didxB, rowsB, semB)
      finish(sidxA, didxA, rowsA, semA)
      start(j + 2, sidxA, didxA, rowsA, semA)
      finish(sidxB, didxB, rowsB, semB)

    finish(sidxA, didxA, rowsA, semA)
    plsc.subcore_barrier()
    pltpu.sync_copy(acc.at[pl.ds(s * RPT, RPT)],
                    out_hbm.at[c].at[pl.ds(s * RPT, RPT)])

  return k(h, src, dst, zeros_nd)


def _sc_classify(h, si, di):
  @functools.partial(
      pl.kernel,
      out_type=jax.ShapeDtypeStruct((LP,), jnp.float32),
      mesh=_mesh(),
      compiler_params=_sc_params(),
      scratch_types=[
          pltpu.VMEM((CC,), jnp.int32),
          pltpu.VMEM((CC,), jnp.int32),
          pltpu.VMEM((CC, D), jnp.float32),
          pltpu.VMEM((CC, D), jnp.float32),
          pltpu.VMEM((CC,), jnp.int32),
          pltpu.VMEM((CC,), jnp.int32),
          pltpu.VMEM((CC, D), jnp.float32),
          pltpu.VMEM((CC, D), jnp.float32),
          pltpu.VMEM((CC,), jnp.float32),
          pltpu.VMEM((16 * 17,), jnp.float32),
          pltpu.SemaphoreType.DMA,
          pltpu.SemaphoreType.DMA,
      ],
  )
  def k(h_hbm, si_hbm, di_hbm, out_hbm,
        sidxA, didxA, arowsA, browsA,
        sidxB, didxB, arowsB, browsB,
        ovec, tile17, semA, semB):
    c = lax.axis_index("c")
    s = lax.axis_index("s")
    wid = c * NS + s
    lanes = lax.iota(jnp.int32, 16)

    def start(j, sidx, didx, arows, brows, sem):
      base = (wid * CPW + j) * CC
      pltpu.sync_copy(si_hbm.at[pl.ds(base, CC)], sidx)
      pltpu.sync_copy(di_hbm.at[pl.ds(base, CC)], didx)
      pltpu.async_copy(h_hbm.at[sidx], arows, sem)
      pltpu.async_copy(h_hbm.at[didx], brows, sem)

    def wait(sidx, didx, arows, brows, sem):
      pltpu.make_async_copy(h_hbm.at[sidx], arows, sem).wait()
      pltpu.make_async_copy(h_hbm.at[didx], brows, sem).wait()

    def compute(j, arows, brows):
      @pl.loop(0, CC, step=16)
      def _(p0):
        for p in range(16):
          pi = p0 + p
          acc = arows[pi, pl.ds(0, 16)] * brows[pi, pl.ds(0, 16)]
          for q in range(1, D // 16):
            acc += arows[pi, pl.ds(q * 16, 16)] * brows[pi, pl.ds(q * 16, 16)]
          plsc.store_scatter(tile17, [lanes + 17 * p], acc)
        sums = plsc.load_gather(tile17, [lanes * 17])
        for kcol in range(1, 16):
          sums += plsc.load_gather(tile17, [lanes * 17 + kcol])
        ovec[pl.ds(p0, 16)] = sums

      pltpu.sync_copy(ovec, out_hbm.at[pl.ds((wid * CPW + j) * CC, CC)])

    start(0, sidxA, didxA, arowsA, browsA, semA)

    @pl.loop(0, CPW - 1, step=2)
    def _(j):
      start(j + 1, sidxB, didxB, arowsB, browsB, semB)
      wait(sidxA, didxA, arowsA, browsA, semA)
      compute(j, arowsA, browsA)
      start(j + 2, sidxA, didxA, arowsA, browsA, semA)
      wait(sidxB, didxB, arowsB, browsB, semB)
      compute(j + 1, arowsB, browsB)

    wait(sidxA, didxA, arowsA, browsA, semA)
    compute(CPW - 1, arowsA, browsA)

  return k(h, si, di)


def _tc_encode(x, w_t, b, emb):
  BM = 1000

  def body(x_ref, w_ref, b_ref, e_ref, o_ref):
    o_ref[...] = (
        jax.lax.dot(x_ref[...], w_ref[...],
                    precision=lax.Precision.HIGHEST,
                    preferred_element_type=jnp.float32)
        + b_ref[...] + e_ref[...])

  return pl.pallas_call(
      body,
      grid=(N // BM,),
      in_specs=[
          pl.BlockSpec((BM, D), lambda i: (i, 0)),
          pl.BlockSpec((D, D), lambda i: (0, 0)),
          pl.BlockSpec((1, D), lambda i: (0, 0)),
          pl.BlockSpec((BM, D), lambda i: (i, 0)),
      ],
      out_specs=pl.BlockSpec((BM, D), lambda i: (i, 0)),
      out_shape=jax.ShapeDtypeStruct((N, D), jnp.float32),
  )(x, w_t, b, emb)


def _tc_layer(p, degp, h_prev, wl_t, wr_t, bl, relu):
  BM = 1000

  def body(p_ref, d_ref, h_ref, wl_ref, wr_ref, b_ref, o_ref):
    agg = p_ref[0] + p_ref[1]
    deg = d_ref[0, :, 0:1] + d_ref[1, :, 0:1]
    mean = agg / jnp.maximum(deg, 1.0)
    out = (
        jax.lax.dot(mean, wl_ref[...], precision=lax.Precision.HIGHEST,
                    preferred_element_type=jnp.float32)
        + jax.lax.dot(h_ref[...], wr_ref[...],
                      precision=lax.Precision.HIGHEST,
                      preferred_element_type=jnp.float32)
        + b_ref[...])
    if relu:
      out = jnp.maximum(out, 0.0)
    o_ref[...] = out

  return pl.pallas_call(
      body,
      grid=(N // BM,),
      in_specs=[
          pl.BlockSpec((NC, BM, D), lambda i: (0, i, 0)),
          pl.BlockSpec((NC, BM, 16), lambda i: (0, i, 0)),
          pl.BlockSpec((BM, D), lambda i: (i, 0)),
          pl.BlockSpec((D, D), lambda i: (0, 0)),
          pl.BlockSpec((D, D), lambda i: (0, 0)),
          pl.BlockSpec((1, D), lambda i: (0, 0)),
      ],
      out_specs=pl.BlockSpec((BM, D), lambda i: (i, 0)),
      out_shape=jax.ShapeDtypeStruct((N, D), jnp.float32),
  )(p, degp, h_prev, wl_t, wr_t, bl)


def kernel(x, W_lin, b_lin, emb, Wl1, bl1, Wr1, Wl2, bl2, Wr2,
           edge_index, node_ids, edge_label_index):
  epad = E2 - E
  src = jnp.concatenate([edge_index[0].astype(jnp.int32),
                         (jnp.arange(epad, dtype=jnp.int32) * 97) % N])
  dst = jnp.concatenate([edge_index[1].astype(jnp.int32),
                         N + (jnp.arange(epad, dtype=jnp.int32) % (N2 - N))])
  eli = edge_label_index.astype(jnp.int32)
  padv = (jnp.arange(LP - L, dtype=jnp.int32) * 97) % N
  si = jnp.concatenate([eli[0], padv])
  di = jnp.concatenate([eli[1], padv])

  zeros_nd = jnp.zeros((N2, D), jnp.float32)

  degp = _sc_degree(dst)
  h0 = _tc_encode(x, W_lin.T, b_lin.reshape(1, D), emb)

  p1 = _sc_aggregate(h0, src, dst, zeros_nd)
  h1 = _tc_layer(p1, degp, h0, Wl1.T, Wr1.T, bl1.reshape(1, D), relu=True)

  p2 = _sc_aggregate(h1, src, dst, zeros_nd)
  h2 = _tc_layer(p2, degp, h1, Wl2.T, Wr2.T, bl2.reshape(1, D), relu=False)

  pred = _sc_classify(h2, si, di)
  return pred[:L]


# speedup vs baseline: 9.0769x; 1.1142x over previous
import dataclasses
import functools

import jax
import jax.numpy as jnp
from jax import lax
from jax.experimental import pallas as pl
from jax.experimental.pallas import tpu as pltpu
from jax.experimental.pallas import tpu_sc as plsc

N = 10000
E = 320000
L = 100000
D = 128

NC = 2
NS = 16
NW = NC * NS

N2 = 10240
RPT = N2 // NS
EC = 128
ECHUNKS = 79
NBUF = 2
EPW = EC * ECHUNKS
E2 = EPW * NW

CC = 128
CPW = 25
LP = NW * CPW * CC

_mesh = functools.partial(
    plsc.VectorSubcoreMesh, core_axis_name="c", subcore_axis_name="s")


def _sc_params():
  cp = pltpu.CompilerParams()
  if "needs_layout_passes" in pltpu.CompilerParams.__dataclass_fields__:
    cp = dataclasses.replace(cp, needs_layout_passes=False)
  return cp


def _sc_degree(dst):
  @functools.partial(
      pl.kernel,
      out_type=jax.ShapeDtypeStruct((NC, N2, 16), jnp.float32),
      mesh=_mesh(),
      compiler_params=_sc_params(),
      scratch_types=[
          pltpu.VMEM((EPW,), jnp.int32),
          pltpu.VMEM((80, D), jnp.float32),
          pltpu.VMEM((80,), jnp.int32),
          pltpu.VMEM((RPT // D, D), jnp.float32),
          pltpu.VMEM((RPT, 16), jnp.float32),
          pltpu.VMEM_SHARED((80, D), jnp.float32),
      ],
  )
  def k(dst_hbm, out_hbm, didx, tab, rowids, degv, rowbuf, spacc):
    c = lax.axis_index("c")
    s = lax.axis_index("s")
    wid = c * NS + s
    z16 = jnp.zeros((16,), jnp.float32)
    pltpu.sync_copy(dst_hbm.at[pl.ds(wid * EPW, EPW)], didx)

    @pl.loop(0, 80)
    def _(i):
      for q in range(D // 16):
        tab[i, pl.ds(q * 16, 16)] = z16

    @pl.loop(0, 80, step=16)
    def _(i):
      rowids[pl.ds(i, 16)] = lax.iota(jnp.int32, 16) + i

    @pl.loop(0, RPT // D)
    def _(i):
      for q in range(D // 16):
        degv[i, pl.ds(q * 16, 16)] = z16

    pltpu.sync_copy(degv, spacc.at[pl.ds(s * (RPT // D), RPT // D)])
    plsc.subcore_barrier()

    ones = jnp.ones((16,), jnp.float32)

    @pl.loop(0, EPW, step=16 * 4)
    def _(j):
      for g in range(4):
        d = didx[pl.ds(j + g * 16, 16)]
        plsc.addupdate_scatter(
            tab, [lax.shift_right_logical(d, 7), d & 127], ones)

    pltpu.sync_copy(tab, spacc.at[rowids], add=True)
    plsc.subcore_barrier()

    pltpu.sync_copy(spacc.at[pl.ds(s * (RPT // D), RPT // D)], degv)

    @pl.loop(0, RPT)
    def _(i):
      iv = jnp.full((16,), i, jnp.int32)
      rowbuf[i, pl.ds(0, 16)] = plsc.load_gather(
          degv, [lax.shift_right_logical(iv, 7), iv & 127])

    pltpu.sync_copy(rowbuf, out_hbm.at[c].at[pl.ds(s * RPT, RPT)])

  return k(dst)


def _sc_aggregate(h, il, zeros_nd):
  @functools.partial(
      pl.kernel,
      out_type=jax.ShapeDtypeStruct((NC, N2, D), jnp.float32),
      mesh=_mesh(),
      scratch_types=[
          pltpu.VMEM((2, EC), jnp.int32),
          pltpu.VMEM((EC, D), jnp.float32),
          pltpu.VMEM((2, EC), jnp.int32),
          pltpu.VMEM((EC, D), jnp.float32),
          pltpu.VMEM_SHARED((N2, D), jnp.float32),
          pltpu.SemaphoreType.DMA,
          pltpu.SemaphoreType.DMA,
      ],
  )
  def k(h_hbm, il_hbm, z_hbm, out_hbm,
        idxA, rowsA, idxB, rowsB, acc, semA, semB):
    c = lax.axis_index("c")
    s = lax.axis_index("s")
    wid = c * NS + s

    def start(j, idx, rows, sem):
      pltpu.sync_copy(il_hbm.at[wid * ECHUNKS + j], idx)
      pltpu.async_copy(h_hbm.at[idx.at[0]], rows, sem)

    def finish(idx, rows, sem):
      pltpu.make_async_copy(h_hbm.at[idx.at[0]], rows, sem).wait()
      pltpu.sync_copy(rows, acc.at[idx.at[1]], add=True)

    start(0, idxA, rowsA, semA)
    pltpu.sync_copy(z_hbm.at[pl.ds(s * RPT, RPT)], acc.at[pl.ds(s * RPT, RPT)])
    plsc.subcore_barrier()

    @pl.loop(0, ECHUNKS - 1, step=2)
    def _(j):
      start(j + 1, idxB, rowsB, semB)
      finish(idxA, rowsA, semA)
      start(j + 2, idxA, rowsA, semA)
      finish(idxB, rowsB, semB)

    finish(idxA, rowsA, semA)
    plsc.subcore_barrier()
    pltpu.sync_copy(acc.at[pl.ds(s * RPT, RPT)],
                    out_hbm.at[c].at[pl.ds(s * RPT, RPT)])

  return k(h, il, zeros_nd)


def _sc_classify(h, si, di):
  @functools.partial(
      pl.kernel,
      out_type=jax.ShapeDtypeStruct((LP,), jnp.float32),
      mesh=_mesh(),
      compiler_params=_sc_params(),
      scratch_types=[
          pltpu.VMEM((CC,), jnp.int32),
          pltpu.VMEM((CC,), jnp.int32),
          pltpu.VMEM((CC, D), jnp.float32),
          pltpu.VMEM((CC, D), jnp.float32),
          pltpu.VMEM((CC,), jnp.int32),
          pltpu.VMEM((CC,), jnp.int32),
          pltpu.VMEM((CC, D), jnp.float32),
          pltpu.VMEM((CC, D), jnp.float32),
          pltpu.VMEM((CC,), jnp.float32),
          pltpu.VMEM((16 * 17,), jnp.float32),
          pltpu.SemaphoreType.DMA,
          pltpu.SemaphoreType.DMA,
      ],
  )
  def k(h_hbm, si_hbm, di_hbm, out_hbm,
        sidxA, didxA, arowsA, browsA,
        sidxB, didxB, arowsB, browsB,
        ovec, tile17, semA, semB):
    c = lax.axis_index("c")
    s = lax.axis_index("s")
    wid = c * NS + s
    lanes = lax.iota(jnp.int32, 16)

    def start(j, sidx, didx, arows, brows, sem):
      base = (wid * CPW + j) * CC
      pltpu.sync_copy(si_hbm.at[pl.ds(base, CC)], sidx)
      pltpu.sync_copy(di_hbm.at[pl.ds(base, CC)], didx)
      pltpu.async_copy(h_hbm.at[sidx], arows, sem)
      pltpu.async_copy(h_hbm.at[didx], brows, sem)

    def wait(sidx, didx, arows, brows, sem):
      pltpu.make_async_copy(h_hbm.at[sidx], arows, sem).wait()
      pltpu.make_async_copy(h_hbm.at[didx], brows, sem).wait()

    def compute(j, arows, brows):
      @pl.loop(0, CC, step=16)
      def _(p0):
        for p in range(16):
          pi = p0 + p
          acc = arows[pi, pl.ds(0, 16)] * brows[pi, pl.ds(0, 16)]
          for q in range(1, D // 16):
            acc += arows[pi, pl.ds(q * 16, 16)] * brows[pi, pl.ds(q * 16, 16)]
          plsc.store_scatter(tile17, [lanes + 17 * p], acc)
        cols = [plsc.load_gather(tile17, [lanes * 17 + kcol])
                for kcol in range(16)]
        while len(cols) > 1:
          cols = [a + b for a, b in zip(cols[::2], cols[1::2])]
        ovec[pl.ds(p0, 16)] = cols[0]

      pltpu.sync_copy(ovec, out_hbm.at[pl.ds((wid * CPW + j) * CC, CC)])

    start(0, sidxA, didxA, arowsA, browsA, semA)

    @pl.loop(0, CPW - 1, step=2)
    def _(j):
      start(j + 1, sidxB, didxB, arowsB, browsB, semB)
      wait(sidxA, didxA, arowsA, browsA, semA)
      compute(j, arowsA, browsA)
      start(j + 2, sidxA, didxA, arowsA, browsA, semA)
      wait(sidxB, didxB, arowsB, browsB, semB)
      compute(j + 1, arowsB, browsB)

    wait(sidxA, didxA, arowsA, browsA, semA)
    compute(CPW - 1, arowsA, browsA)

  return k(h, si, di)


def _tc_encode(x, w_t, b, emb):
  BM = 1000

  def body(x_ref, w_ref, b_ref, e_ref, o_ref):
    o_ref[...] = (
        jax.lax.dot(x_ref[...], w_ref[...],
                    precision=lax.Precision.HIGHEST,
                    preferred_element_type=jnp.float32)
        + b_ref[...] + e_ref[...])

  return pl.pallas_call(
      body,
      grid=(N // BM,),
      in_specs=[
          pl.BlockSpec((BM, D), lambda i: (i, 0)),
          pl.BlockSpec((D, D), lambda i: (0, 0)),
          pl.BlockSpec((1, D), lambda i: (0, 0)),
          pl.BlockSpec((BM, D), lambda i: (i, 0)),
      ],
      out_specs=pl.BlockSpec((BM, D), lambda i: (i, 0)),
      out_shape=jax.ShapeDtypeStruct((N, D), jnp.float32),
  )(x, w_t, b, emb)


def _tc_layer(p, degp, h_prev, wl_t, wr_t, bl, relu):
  BM = 1000

  def body(p_ref, d_ref, h_ref, wl_ref, wr_ref, b_ref, o_ref):
    agg = p_ref[0] + p_ref[1]
    deg = d_ref[0, :, 0:1] + d_ref[1, :, 0:1]
    mean = agg / jnp.maximum(deg, 1.0)
    out = (
        jax.lax.dot(mean, wl_ref[...], precision=lax.Precision.HIGHEST,
                    preferred_element_type=jnp.float32)
        + jax.lax.dot(h_ref[...], wr_ref[...],
                      precision=lax.Precision.HIGHEST,
                      preferred_element_type=jnp.float32)
        + b_ref[...])
    if relu:
      out = jnp.maximum(out, 0.0)
    o_ref[...] = out

  return pl.pallas_call(
      body,
      grid=(N // BM,),
      in_specs=[
          pl.BlockSpec((NC, BM, D), lambda i: (0, i, 0)),
          pl.BlockSpec((NC, BM, 16), lambda i: (0, i, 0)),
          pl.BlockSpec((BM, D), lambda i: (i, 0)),
          pl.BlockSpec((D, D), lambda i: (0, 0)),
          pl.BlockSpec((D, D), lambda i: (0, 0)),
          pl.BlockSpec((1, D), lambda i: (0, 0)),
      ],
      out_specs=pl.BlockSpec((BM, D), lambda i: (i, 0)),
      out_shape=jax.ShapeDtypeStruct((N, D), jnp.float32),
  )(p, degp, h_prev, wl_t, wr_t, bl)


def kernel(x, W_lin, b_lin, emb, Wl1, bl1, Wr1, Wl2, bl2, Wr2,
           edge_index, node_ids, edge_label_index):
  epad = E2 - E
  src = jnp.concatenate([edge_index[0].astype(jnp.int32),
                         (jnp.arange(epad, dtype=jnp.int32) * 97) % N])
  dst = jnp.concatenate([edge_index[1].astype(jnp.int32),
                         N + (jnp.arange(epad, dtype=jnp.int32) % (N2 - N))])
  eli = edge_label_index.astype(jnp.int32)
  padv = (jnp.arange(LP - L, dtype=jnp.int32) * 97) % N
  si = jnp.concatenate([eli[0], padv])
  di = jnp.concatenate([eli[1], padv])

  zeros_nd = jnp.zeros((N2, D), jnp.float32)

  il = jnp.stack([src.reshape(-1, EC), dst.reshape(-1, EC)], axis=1)

  degp = _sc_degree(dst)
  h0 = _tc_encode(x, W_lin.T, b_lin.reshape(1, D), emb)

  p1 = _sc_aggregate(h0, il, zeros_nd)
  h1 = _tc_layer(p1, degp, h0, Wl1.T, Wr1.T, bl1.reshape(1, D), relu=True)

  p2 = _sc_aggregate(h1, il, zeros_nd)
  h2 = _tc_layer(p2, degp, h1, Wl2.T, Wr2.T, bl2.reshape(1, D), relu=False)

  pred = _sc_classify(h2, si, di)
  return pred[:L]


# speedup vs baseline: 9.2735x; 1.0217x over previous
import dataclasses
import functools

import jax
import jax.numpy as jnp
from jax import lax
from jax.experimental import pallas as pl
from jax.experimental.pallas import tpu as pltpu
from jax.experimental.pallas import tpu_sc as plsc

N = 10000
E = 320000
L = 100000
D = 128

NC = 2
NS = 16
NW = NC * NS

N2 = 10240
RPT = N2 // NS
EC = 128
ECHUNKS = 79
NBUF = 2
EPW = EC * ECHUNKS
E2 = EPW * NW

CC = 128
CPW = 25
LP = NW * CPW * CC

_mesh = functools.partial(
    plsc.VectorSubcoreMesh, core_axis_name="c", subcore_axis_name="s")


def _sc_params():
  cp = pltpu.CompilerParams()
  if "needs_layout_passes" in pltpu.CompilerParams.__dataclass_fields__:
    cp = dataclasses.replace(cp, needs_layout_passes=False)
  return cp


def _sc_degree(dst):
  @functools.partial(
      pl.kernel,
      out_type=jax.ShapeDtypeStruct((NC, N2, 16), jnp.float32),
      mesh=_mesh(),
      compiler_params=_sc_params(),
      scratch_types=[
          pltpu.VMEM((EPW,), jnp.int32),
          pltpu.VMEM((80, D), jnp.float32),
          pltpu.VMEM((80,), jnp.int32),
          pltpu.VMEM((RPT // D, D), jnp.float32),
          pltpu.VMEM((RPT, 16), jnp.float32),
          pltpu.VMEM_SHARED((80, D), jnp.float32),
      ],
  )
  def k(dst_hbm, out_hbm, didx, tab, rowids, degv, rowbuf, spacc):
    c = lax.axis_index("c")
    s = lax.axis_index("s")
    wid = c * NS + s
    z16 = jnp.zeros((16,), jnp.float32)
    pltpu.sync_copy(dst_hbm.at[pl.ds(wid * EPW, EPW)], didx)

    @pl.loop(0, 80)
    def _(i):
      for q in range(D // 16):
        tab[i, pl.ds(q * 16, 16)] = z16

    @pl.loop(0, 80, step=16)
    def _(i):
      rowids[pl.ds(i, 16)] = lax.iota(jnp.int32, 16) + i

    @pl.loop(0, RPT // D)
    def _(i):
      for q in range(D // 16):
        degv[i, pl.ds(q * 16, 16)] = z16

    pltpu.sync_copy(degv, spacc.at[pl.ds(s * (RPT // D), RPT // D)])
    plsc.subcore_barrier()

    ones = jnp.ones((16,), jnp.float32)

    @pl.loop(0, EPW, step=16 * 4)
    def _(j):
      for g in range(4):
        d = didx[pl.ds(j + g * 16, 16)]
        plsc.addupdate_scatter(
            tab, [lax.shift_right_logical(d, 7), d & 127], ones)

    pltpu.sync_copy(tab, spacc.at[rowids], add=True)
    plsc.subcore_barrier()

    pltpu.sync_copy(spacc.at[pl.ds(s * (RPT // D), RPT // D)], degv)

    @pl.loop(0, RPT)
    def _(i):
      iv = jnp.full((16,), i, jnp.int32)
      rowbuf[i, pl.ds(0, 16)] = plsc.load_gather(
          degv, [lax.shift_right_logical(iv, 7), iv & 127])

    pltpu.sync_copy(rowbuf, out_hbm.at[c].at[pl.ds(s * RPT, RPT)])

  return k(dst)


def _sc_aggregate(h, il, zeros_nd):
  @functools.partial(
      pl.kernel,
      out_type=jax.ShapeDtypeStruct((NC, N2, D), jnp.float32),
      mesh=_mesh(),
      scratch_types=[
          pltpu.VMEM((2, EC), jnp.int32),
          pltpu.VMEM((EC, D), jnp.float32),
          pltpu.VMEM((2, EC), jnp.int32),
          pltpu.VMEM((EC, D), jnp.float32),
          pltpu.VMEM_SHARED((N2, D), jnp.float32),
          pltpu.SemaphoreType.DMA,
          pltpu.SemaphoreType.DMA,
      ],
  )
  def k(h_hbm, il_hbm, z_hbm, out_hbm,
        idxA, rowsA, idxB, rowsB, acc, semA, semB):
    c = lax.axis_index("c")
    s = lax.axis_index("s")
    wid = c * NS + s

    def start(j, idx, rows, sem):
      pltpu.sync_copy(il_hbm.at[wid * ECHUNKS + j], idx)
      pltpu.async_copy(h_hbm.at[idx.at[0]], rows, sem)

    def finish(idx, rows, sem):
      pltpu.make_async_copy(h_hbm.at[idx.at[0]], rows, sem).wait()
      pltpu.sync_copy(rows, acc.at[idx.at[1]], add=True)

    start(0, idxA, rowsA, semA)
    pltpu.sync_copy(z_hbm.at[pl.ds(s * RPT, RPT)], acc.at[pl.ds(s * RPT, RPT)])
    plsc.subcore_barrier()

    @pl.loop(0, ECHUNKS - 1, step=2)
    def _(j):
      start(j + 1, idxB, rowsB, semB)
      finish(idxA, rowsA, semA)
      start(j + 2, idxA, rowsA, semA)
      finish(idxB, rowsB, semB)

    finish(idxA, rowsA, semA)
    plsc.subcore_barrier()
    pltpu.sync_copy(acc.at[pl.ds(s * RPT, RPT)],
                    out_hbm.at[c].at[pl.ds(s * RPT, RPT)])

  return k(h, il, zeros_nd)


def _sc_classify(h, ilc):
  @functools.partial(
      pl.kernel,
      out_type=jax.ShapeDtypeStruct((LP,), jnp.float32),
      mesh=_mesh(),
      compiler_params=_sc_params(),
      scratch_types=[
          pltpu.VMEM((2, CC), jnp.int32),
          pltpu.VMEM((CC, D), jnp.float32),
          pltpu.VMEM((CC, D), jnp.float32),
          pltpu.VMEM((2, CC), jnp.int32),
          pltpu.VMEM((CC, D), jnp.float32),
          pltpu.VMEM((CC, D), jnp.float32),
          pltpu.VMEM((CC,), jnp.float32),
          pltpu.VMEM((16 * 17,), jnp.float32),
          pltpu.SemaphoreType.DMA,
          pltpu.SemaphoreType.DMA,
      ],
  )
  def k(h_hbm, ilc_hbm, out_hbm,
        idxA, arowsA, browsA, idxB, arowsB, browsB,
        ovec, tile17, semA, semB):
    c = lax.axis_index("c")
    s = lax.axis_index("s")
    wid = c * NS + s
    lanes = lax.iota(jnp.int32, 16)

    def start(j, idx, arows, brows, sem):
      pltpu.sync_copy(ilc_hbm.at[wid * CPW + j], idx)
      pltpu.async_copy(h_hbm.at[idx.at[0]], arows, sem)
      pltpu.async_copy(h_hbm.at[idx.at[1]], brows, sem)

    def wait(idx, arows, brows, sem):
      pltpu.make_async_copy(h_hbm.at[idx.at[0]], arows, sem).wait()
      pltpu.make_async_copy(h_hbm.at[idx.at[1]], brows, sem).wait()

    def compute(j, arows, brows):
      @pl.loop(0, CC, step=16)
      def _(p0):
        for p in range(16):
          pi = p0 + p
          acc = arows[pi, pl.ds(0, 16)] * brows[pi, pl.ds(0, 16)]
          for q in range(1, D // 16):
            acc += arows[pi, pl.ds(q * 16, 16)] * brows[pi, pl.ds(q * 16, 16)]
          plsc.store_scatter(tile17, [lanes + 17 * p], acc)
        cols = [plsc.load_gather(tile17, [lanes * 17 + kcol])
                for kcol in range(16)]
        while len(cols) > 1:
          cols = [a + b for a, b in zip(cols[::2], cols[1::2])]
        ovec[pl.ds(p0, 16)] = cols[0]

      pltpu.sync_copy(ovec, out_hbm.at[pl.ds((wid * CPW + j) * CC, CC)])

    start(0, idxA, arowsA, browsA, semA)

    @pl.loop(0, CPW - 1, step=2)
    def _(j):
      start(j + 1, idxB, arowsB, browsB, semB)
      wait(idxA, arowsA, browsA, semA)
      compute(j, arowsA, browsA)
      start(j + 2, idxA, arowsA, browsA, semA)
      wait(idxB, arowsB, browsB, semB)
      compute(j + 1, arowsB, browsB)

    wait(idxA, arowsA, browsA, semA)
    compute(CPW - 1, arowsA, browsA)

  return k(h, ilc)


def _tc_encode(x, w_t, b, emb):
  BM = 1000

  def body(x_ref, w_ref, b_ref, e_ref, o_ref):
    o_ref[...] = (
        jax.lax.dot(x_ref[...], w_ref[...],
                    precision=lax.Precision.HIGHEST,
                    preferred_element_type=jnp.float32)
        + b_ref[...] + e_ref[...])

  return pl.pallas_call(
      body,
      grid=(N // BM,),
      in_specs=[
          pl.BlockSpec((BM, D), lambda i: (i, 0)),
          pl.BlockSpec((D, D), lambda i: (0, 0)),
          pl.BlockSpec((1, D), lambda i: (0, 0)),
          pl.BlockSpec((BM, D), lambda i: (i, 0)),
      ],
      out_specs=pl.BlockSpec((BM, D), lambda i: (i, 0)),
      out_shape=jax.ShapeDtypeStruct((N, D), jnp.float32),
  )(x, w_t, b, emb)


def _tc_layer(p, degp, h_prev, wl_t, wr_t, bl, relu):
  BM = 1000

  def body(p_ref, d_ref, h_ref, wl_ref, wr_ref, b_ref, o_ref):
    agg = p_ref[0] + p_ref[1]
    deg = d_ref[0, :, 0:1] + d_ref[1, :, 0:1]
    mean = agg / jnp.maximum(deg, 1.0)
    out = (
        jax.lax.dot(mean, wl_ref[...], precision=lax.Precision.HIGHEST,
                    preferred_element_type=jnp.float32)
        + jax.lax.dot(h_ref[...], wr_ref[...],
                      precision=lax.Precision.HIGHEST,
                      preferred_element_type=jnp.float32)
        + b_ref[...])
    if relu:
      out = jnp.maximum(out, 0.0)
    o_ref[...] = out

  return pl.pallas_call(
      body,
      grid=(N // BM,),
      in_specs=[
          pl.BlockSpec((NC, BM, D), lambda i: (0, i, 0)),
          pl.BlockSpec((NC, BM, 16), lambda i: (0, i, 0)),
          pl.BlockSpec((BM, D), lambda i: (i, 0)),
          pl.BlockSpec((D, D), lambda i: (0, 0)),
          pl.BlockSpec((D, D), lambda i: (0, 0)),
          pl.BlockSpec((1, D), lambda i: (0, 0)),
      ],
      out_specs=pl.BlockSpec((BM, D), lambda i: (i, 0)),
      out_shape=jax.ShapeDtypeStruct((N, D), jnp.float32),
  )(p, degp, h_prev, wl_t, wr_t, bl)


def kernel(x, W_lin, b_lin, emb, Wl1, bl1, Wr1, Wl2, bl2, Wr2,
           edge_index, node_ids, edge_label_index):
  epad = E2 - E
  src = jnp.concatenate([edge_index[0].astype(jnp.int32),
                         (jnp.arange(epad, dtype=jnp.int32) * 97) % N])
  dst = jnp.concatenate([edge_index[1].astype(jnp.int32),
                         N + (jnp.arange(epad, dtype=jnp.int32) % (N2 - N))])
  eli = edge_label_index.astype(jnp.int32)
  padv = (jnp.arange(LP - L, dtype=jnp.int32) * 97) % N
  si = jnp.concatenate([eli[0], padv])
  di = jnp.concatenate([eli[1], padv])
  ilc = jnp.stack([si.reshape(-1, CC), di.reshape(-1, CC)], axis=1)

  zeros_nd = jnp.zeros((N2, D), jnp.float32)

  il = jnp.stack([src.reshape(-1, EC), dst.reshape(-1, EC)], axis=1)

  degp = _sc_degree(dst)
  h0 = _tc_encode(x, W_lin.T, b_lin.reshape(1, D), emb)

  p1 = _sc_aggregate(h0, il, zeros_nd)
  h1 = _tc_layer(p1, degp, h0, Wl1.T, Wr1.T, bl1.reshape(1, D), relu=True)

  p2 = _sc_aggregate(h1, il, zeros_nd)
  h2 = _tc_layer(p2, degp, h1, Wl2.T, Wr2.T, bl2.reshape(1, D), relu=False)

  pred = _sc_classify(h2, ilc)
  return pred[:L]
